# baseline (speedup 1.0000x reference)
import functools

import jax
import jax.numpy as jnp
from jax import lax
from jax.experimental import pallas as pl
from jax.experimental.pallas import tpu as pltpu
from jax.experimental.pallas import tpu_sc as plsc

B, N, D, K = 16, 1024, 32, 8192
T = B * N
TOK_TILE = 512
KC = 2048
LANES = 128

NC = 2
NS = 16
NW = NC * NS
T_PER_W = T // (2 * NW)
GCH = 128
N_CH = T_PER_W // GCH


N_SPLIT = 2
TSPLIT = T // N_SPLIT
N_TILES = TSPLIT // TOK_TILE
GRID_N = N_TILES + 1


def _tc_index_body(x_ref, e_ref, idx_ref, bb_ref, bt_ref):
    i = pl.program_id(0)

    @pl.when(i < N_TILES)
    def _compute():
        x = x_ref[...]
        n = jnp.sqrt(jnp.sum(x * x, axis=1, keepdims=True))
        xn = x / jnp.maximum(n, 1e-12)

        best = jnp.full((TOK_TILE, LANES), -jnp.inf, jnp.float32)
        btile = jnp.zeros((TOK_TILE, LANES), jnp.float32)
        for j in range(K // KC):
            e = e_ref[pl.ds(j * KC, KC), :]
            d = lax.dot_general(xn, e, (((1,), (1,)), ((), ())),
                                preferred_element_type=jnp.float32)
            for s in range(KC // LANES):
                ds = lax.slice(d, (0, s * LANES), (TOK_TILE, (s + 1) * LANES))
                gt = ds > best
                best = jnp.maximum(best, ds)
                btile = jnp.where(gt, jnp.float32(j * (KC // LANES) + s), btile)
        bb_ref[i % 2] = best
        bt_ref[i % 2] = btile

    @pl.when(i > 0)
    def _finale():
        best = bb_ref[(i + 1) % 2]
        btile = bt_ref[(i + 1) % 2]
        lane_iota = lax.broadcasted_iota(
            jnp.int32, (TOK_TILE, LANES), 1).astype(jnp.float32)
        bidx = btile * LANES + lane_iota
        m = jnp.max(best, axis=1, keepdims=True)
        cand = jnp.where(best == m, bidx, jnp.float32(K))
        idx_ref[...] = jnp.min(cand, axis=1).astype(jnp.int32)


_tc_indices = pl.pallas_call(
    _tc_index_body,
    grid=(GRID_N,),
    in_specs=[
        pl.BlockSpec((TOK_TILE, D), lambda i: (jnp.minimum(i, N_TILES - 1), 0)),
        pl.BlockSpec((K, D), lambda i: (0, 0)),
    ],
    out_specs=pl.BlockSpec((TOK_TILE,), lambda i: ((i + GRID_N - 1) % GRID_N,)),
    out_shape=jax.ShapeDtypeStruct((GRID_N * TOK_TILE,), jnp.int32),
    scratch_shapes=[
        pltpu.VMEM((2, TOK_TILE, LANES), jnp.float32),
        pltpu.VMEM((2, TOK_TILE, LANES), jnp.float32),
    ],
)


@functools.lru_cache(maxsize=1)
def _make_sc_gather():
    @functools.partial(
        pl.kernel,
        mesh=plsc.VectorSubcoreMesh(core_axis_name="c", subcore_axis_name="s"),
        out_type=jax.ShapeDtypeStruct((NW, N_CH, GCH, D), jnp.float32),
        scratch_types=[
            pltpu.VMEM((N_CH, GCH), jnp.int32),
            pltpu.VMEM((N_CH, GCH, D), jnp.float32),
            pltpu.SemaphoreType.DMA,
        ],
        compiler_params=pltpu.CompilerParams(use_tc_tiling_on_sc=False),
    )
    def _sc_gather(table_hbm, idx_hbm, out_hbm, idx_v, rows_v, sem):
        wid = lax.axis_index("s") * NC + lax.axis_index("c")
        pltpu.sync_copy(idx_hbm.at[wid], idx_v)
        copies = [
            pltpu.async_copy(table_hbm.at[idx_v.at[j]], rows_v.at[j], sem)
            for j in range(N_CH)
        ]
        for c in copies:
            c.wait()
        pltpu.sync_copy(rows_v, out_hbm.at[wid])

    return _sc_gather


def kernel(x, embed):
    xf = x.reshape(T, D)
    table = embed.reshape(K, D)
    sc_gather = _make_sc_gather()
    parts = []
    for s in range(N_SPLIT):
        xs = lax.slice(xf, (s * TSPLIT, 0), ((s + 1) * TSPLIT, D))
        idx = _tc_indices(xs, table)[:TSPLIT]
        parts.append(sc_gather(table, idx.reshape(NW, N_CH, GCH)))
    rows = jnp.concatenate([p.reshape(TSPLIT, D) for p in parts], axis=0)
    return rows.reshape(B, N, D)


# speedup vs baseline: 1.0537x; 1.0537x over previous
import functools

import jax
import jax.numpy as jnp
from jax import lax
from jax.experimental import pallas as pl
from jax.experimental.pallas import tpu as pltpu
from jax.experimental.pallas import tpu_sc as plsc

B, N, D, K = 16, 1024, 32, 8192
T = B * N
TOK_TILE = 512
KC = 2048
LANES = 128

NC = 2
NS = 16
NW = NC * NS
T_PER_W = T // NW
GCH = 128
N_CH = T_PER_W // GCH


N_TILES = T // TOK_TILE
GRID_N = N_TILES + 1


def _tc_index_body(x_ref, e_ref, idx_ref, bb_ref, bt_ref):
    i = pl.program_id(0)

    @pl.when(i < N_TILES)
    def _compute():
        x = x_ref[...]
        n = jnp.sqrt(jnp.sum(x * x, axis=1, keepdims=True))
        xn = x / jnp.maximum(n, 1e-12)

        acc = []
        half = K // (2 * KC)
        for h in range(2):
            best = jnp.full((TOK_TILE, LANES), -jnp.inf, jnp.float32)
            btile = jnp.zeros((TOK_TILE, LANES), jnp.float32)
            for j in range(h * half, (h + 1) * half):
                e = e_ref[pl.ds(j * KC, KC), :]
                d = lax.dot_general(xn, e, (((1,), (1,)), ((), ())),
                                    preferred_element_type=jnp.float32)
                for s in range(KC // LANES):
                    ds = lax.slice(d, (0, s * LANES),
                                   (TOK_TILE, (s + 1) * LANES))
                    gt = ds > best
                    best = jnp.maximum(best, ds)
                    btile = jnp.where(
                        gt, jnp.float32(j * (KC // LANES) + s), btile)
            acc.append((best, btile))
        (b0, t0), (b1, t1) = acc
        gt = b1 > b0
        bb_ref[i % 2] = jnp.maximum(b0, b1)
        bt_ref[i % 2] = jnp.where(gt, t1, t0)

    @pl.when(i > 0)
    def _finale():
        best = bb_ref[(i + 1) % 2]
        btile = bt_ref[(i + 1) % 2]
        lane_iota = lax.broadcasted_iota(
            jnp.int32, (TOK_TILE, LANES), 1).astype(jnp.float32)
        bidx = btile * LANES + lane_iota
        m = jnp.max(best, axis=1, keepdims=True)
        cand = jnp.where(best == m, bidx, jnp.float32(K))
        idx_ref[...] = jnp.min(cand, axis=1).astype(jnp.int32)


_tc_indices = pl.pallas_call(
    _tc_index_body,
    grid=(GRID_N,),
    in_specs=[
        pl.BlockSpec((TOK_TILE, D), lambda i: (jnp.minimum(i, N_TILES - 1), 0)),
        pl.BlockSpec((K, D), lambda i: (0, 0)),
    ],
    out_specs=pl.BlockSpec((TOK_TILE,), lambda i: ((i + GRID_N - 1) % GRID_N,)),
    out_shape=jax.ShapeDtypeStruct((GRID_N * TOK_TILE,), jnp.int32),
    scratch_shapes=[
        pltpu.VMEM((2, TOK_TILE, LANES), jnp.float32),
        pltpu.VMEM((2, TOK_TILE, LANES), jnp.float32),
    ],
)


@functools.lru_cache(maxsize=1)
def _make_sc_gather():
    @functools.partial(
        pl.kernel,
        mesh=plsc.VectorSubcoreMesh(core_axis_name="c", subcore_axis_name="s"),
        out_type=jax.ShapeDtypeStruct((NW, N_CH, GCH, D), jnp.float32),
        scratch_types=[
            pltpu.VMEM((N_CH, GCH), jnp.int32),
            pltpu.VMEM((N_CH, GCH, D), jnp.float32),
            pltpu.SemaphoreType.DMA,
        ],
        compiler_params=pltpu.CompilerParams(use_tc_tiling_on_sc=False),
    )
    def _sc_gather(table_hbm, idx_hbm, out_hbm, idx_v, rows_v, sem):
        wid = lax.axis_index("s") * NC + lax.axis_index("c")
        pltpu.sync_copy(idx_hbm.at[wid], idx_v)
        copies = [
            pltpu.async_copy(table_hbm.at[idx_v.at[j]], rows_v.at[j], sem)
            for j in range(N_CH)
        ]
        for c in copies:
            c.wait()
        pltpu.sync_copy(rows_v, out_hbm.at[wid])

    return _sc_gather


def kernel(x, embed):
    xf = x.reshape(T, D)
    table = embed.reshape(K, D)
    idx = _tc_indices(xf, table)[:T]
    rows = _make_sc_gather()(table, idx.reshape(NW, N_CH, GCH))
    return rows.reshape(B, N, D)


# speedup vs baseline: 1.0657x; 1.0114x over previous
import functools

import jax
import jax.numpy as jnp
from jax import lax
from jax.experimental import pallas as pl
from jax.experimental.pallas import tpu as pltpu
from jax.experimental.pallas import tpu_sc as plsc

B, N, D, K = 16, 1024, 32, 8192
T = B * N
TOK_TILE = 512
KC = 2048
LANES = 128

NC = 2
NS = 16
NW = NC * NS
T_PER_W = T // NW
GCH = 128
N_CH = T_PER_W // GCH


N_TILES = T // TOK_TILE
GRID_N = N_TILES + 1


def _tc_index_body(x_ref, e_ref, idx_ref, bb_ref, bt_ref):
    i = pl.program_id(0)

    @pl.when(i < N_TILES)
    def _compute():
        x = x_ref[...]
        n = jnp.sqrt(jnp.sum(x * x, axis=1, keepdims=True))
        xn = x / jnp.maximum(n, 1e-12)

        acc = []
        half = K // (2 * KC)
        for h in range(2):
            best = jnp.full((TOK_TILE, LANES), -jnp.inf, jnp.float32)
            btile = jnp.zeros((TOK_TILE, LANES), jnp.float32)
            for j in range(h * half, (h + 1) * half):
                e = e_ref[pl.ds(j * KC, KC), :]
                d = lax.dot_general(xn, e, (((1,), (1,)), ((), ())),
                                    preferred_element_type=jnp.float32)
                for s in range(KC // LANES):
                    ds = lax.slice(d, (0, s * LANES),
                                   (TOK_TILE, (s + 1) * LANES))
                    gt = ds > best
                    best = jnp.maximum(best, ds)
                    btile = jnp.where(
                        gt, jnp.float32(j * (KC // LANES) + s), btile)
            acc.append((best, btile))
        (b0, t0), (b1, t1) = acc
        gt = b1 > b0
        bb_ref[i % 2] = jnp.maximum(b0, b1)
        bt_ref[i % 2] = jnp.where(gt, t1, t0)

    @pl.when(i > 0)
    def _finale():
        best = bb_ref[(i + 1) % 2]
        btile = bt_ref[(i + 1) % 2]
        lane_iota = lax.broadcasted_iota(
            jnp.int32, (TOK_TILE, LANES), 1).astype(jnp.float32)
        bidx = btile * LANES + lane_iota
        m = jnp.max(best, axis=1, keepdims=True)
        cand = jnp.where(best == m, bidx, jnp.float32(K))
        idx_ref[...] = jnp.min(cand, axis=1).astype(jnp.int32)


_tc_indices = pl.pallas_call(
    _tc_index_body,
    grid=(GRID_N,),
    in_specs=[
        pl.BlockSpec((TOK_TILE, D), lambda i: (jnp.minimum(i, N_TILES - 1), 0)),
        pl.BlockSpec((K, D), lambda i: (0, 0)),
    ],
    out_specs=pl.BlockSpec((TOK_TILE,), lambda i: ((i + GRID_N - 1) % GRID_N,)),
    out_shape=jax.ShapeDtypeStruct((GRID_N * TOK_TILE,), jnp.int32),
    scratch_shapes=[
        pltpu.VMEM((2, TOK_TILE, LANES), jnp.float32),
        pltpu.VMEM((2, TOK_TILE, LANES), jnp.float32),
    ],
)


@functools.lru_cache(maxsize=1)
def _make_sc_gather():
    @functools.partial(
        pl.kernel,
        mesh=plsc.VectorSubcoreMesh(core_axis_name="c", subcore_axis_name="s"),
        out_type=jax.ShapeDtypeStruct((NW, N_CH, GCH, D), jnp.float32),
        scratch_types=[
            pltpu.VMEM((T_PER_W,), jnp.int32),
            pltpu.VMEM((N_CH, GCH, D), jnp.float32),
            pltpu.SemaphoreType.DMA,
        ],
        compiler_params=pltpu.CompilerParams(use_tc_tiling_on_sc=False),
    )
    def _sc_gather(table_hbm, idx_hbm, out_hbm, idx_v, rows_v, sem):
        wid = lax.axis_index("s") * NC + lax.axis_index("c")
        pltpu.sync_copy(idx_hbm.at[pl.ds(wid * T_PER_W, T_PER_W)], idx_v)
        copies = [
            pltpu.async_copy(table_hbm.at[idx_v.at[pl.ds(j * GCH, GCH)]],
                             rows_v.at[j], sem)
            for j in range(N_CH)
        ]
        for c in copies:
            c.wait()
        pltpu.sync_copy(rows_v, out_hbm.at[wid])

    return _sc_gather


def kernel(x, embed):
    xf = x.reshape(T, D)
    table = embed.reshape(K, D)
    idx_padded = _tc_indices(xf, table)
    rows = _make_sc_gather()(table, idx_padded)
    return rows.reshape(B, N, D)
